# Initial kernel scaffold; baseline (speedup 1.0000x reference)
import jax
import jax.numpy as jnp
from jax.experimental import pallas as pl


def kernel(x, weight, scales):
    raise NotImplementedError("write your pallas kernel here")


# speedup vs baseline: 4.5217x; 4.5217x over previous
import functools

import jax
import jax.numpy as jnp
import numpy as np
from jax import lax
from jax.experimental import pallas as pl
from jax.experimental.pallas import tpu as pltpu
from jax.experimental.pallas import tpu_sc as plsc

V = 1_000_000
D = 64
B = 4096 * 200
NC, NS, L = 2, 16, 16
NW = NC * NS
BW = B // NW
K = 1024
NCHUNK = BW // K
NIDX = K // 128
TW = 16

_XOR8 = int(np.int32(np.uint32(0x88888888).view(np.int32)))


def _sc_body(x_hbm, t_hbm, out_hbm, idx_v, rows_v, out_v, sem):
    wid = lax.axis_index("s") * NC + lax.axis_index("c")

    lane = lax.iota(jnp.int32, L)
    k7 = lane & 7
    shl = 28 - ((k7 >> 1) << 2) - ((k7 & 1) << 4)
    lo8 = lane < 8

    def chunk_body(c, _):
        gbase = wid * BW + c * K
        row0 = pl.multiple_of(gbase // 128, 8)
        pltpu.sync_copy(x_hbm.at[pl.ds(row0, NIDX)], idx_v)

        cps = []
        for j in range(NIDX):
            cps.append(pltpu.async_copy(
                t_hbm.at[idx_v.at[j]], rows_v.at[pl.ds(j * 128, 128)], sem))
        for cp in cps:
            cp.wait()

        def row_body(r, _):
            wrow = rows_v[r]
            sc = jnp.full((L,), lax.bitcast_convert_type(wrow[8], jnp.float32))
            for v in range(4):
                wab = jnp.where(lo8, wrow[2 * v], wrow[2 * v + 1])
                q = lax.shift_right_arithmetic(lax.shift_left(wab, shl), 28)
                out_v[pl.ds(r * D + v * L, L)] = q.astype(jnp.float32) * sc
            return 0

        lax.fori_loop(0, K, row_body, 0)
        pltpu.sync_copy(out_v, out_hbm.at[pl.ds(gbase * D, K * D)])
        return 0

    lax.fori_loop(0, NCHUNK, chunk_body, 0)


@jax.jit
def _lookup(xf, table):
    mesh = plsc.VectorSubcoreMesh(core_axis_name="c", subcore_axis_name="s")
    run = functools.partial(
        pl.kernel,
        mesh=mesh,
        compiler_params=pltpu.CompilerParams(use_tc_tiling_on_sc=False),
        out_type=jax.ShapeDtypeStruct((B * D,), jnp.float32),
        scratch_types=[
            pltpu.VMEM((NIDX, 128), jnp.int32),
            pltpu.VMEM((K, TW), jnp.int32),
            pltpu.VMEM((K * D,), jnp.float32),
            pltpu.SemaphoreType.DMA,
        ],
    )(_sc_body)
    return run(xf, table)


def kernel(x, weight, scales):
    xf = x.reshape(B // 128, 128)
    sbits = lax.bitcast_convert_type(scales.astype(jnp.float32), jnp.int32)
    table = jnp.concatenate(
        [weight ^ jnp.int32(_XOR8), sbits[:, None],
         jnp.zeros((V, TW - 9), jnp.int32)], axis=1)
    out = _lookup(xf, table)
    return out.reshape(4096, 200, 64)


# speedup vs baseline: 4.9808x; 1.1015x over previous
import functools

import jax
import jax.numpy as jnp
import numpy as np
from jax import lax
from jax.experimental import pallas as pl
from jax.experimental.pallas import tpu as pltpu
from jax.experimental.pallas import tpu_sc as plsc

V = 1_000_000
D = 64
B = 4096 * 200
NC, NS, L = 2, 16, 16
NW = NC * NS
BW = B // NW
K = 512
NP = BW // (2 * K)
KI = K // 128

_XOR8 = int(np.int32(np.uint32(0x88888888).view(np.int32)))


def _sc_body(x_hbm, wp_hbm, sp_hbm, out_hbm,
             ixc_v, ixn_v, ix2_v, ix16_v,
             wA_v, wB_v, sA_v, sB_v, oA_v, oB_v,
             semA, semB, semOA, semOB):
    wid = lax.axis_index("s") * NC + lax.axis_index("c")
    base = wid * BW

    lane = lax.iota(jnp.int32, L)
    k7 = lane & 7
    shl = 28 - ((k7 >> 1) << 2) - ((k7 & 1) << 4)
    wsel = lane >> 3

    def dgather(vec, idx):
        dnums = lax.GatherDimensionNumbers(
            offset_dims=(), collapsed_slice_dims=(0,), start_index_map=(0,))
        return lax.gather(vec, idx[:, None], dnums, slice_sizes=(1,),
                          mode=lax.GatherScatterMode.PROMISE_IN_BOUNDS)

    def stage_pair(p):
        off = pl.multiple_of(base + p * 2 * K, 8)
        pltpu.sync_copy(x_hbm.at[pl.ds(off, 2 * K)], ixn_v)

        def ib(i, _):
            iv = ixn_v[pl.ds(i * L, L)]
            ix2_v[pl.ds(i * L, L)] = iv >> 1
            ix16_v[pl.ds(i * L, L)] = iv >> 4
            return 0
        lax.fori_loop(0, 2 * K // L, ib, 0)

    def fire_gathers(c_half, w_v, s_v, sem):
        for j in range(KI):
            o = c_half * K + j * 128
            pltpu.async_copy(wp_hbm.at[ix2_v.at[pl.ds(o, 128)]],
                             w_v.at[pl.ds(j * 128, 128)], sem)
            pltpu.async_copy(sp_hbm.at[ix16_v.at[pl.ds(o, 128)]],
                             s_v.at[pl.ds(j * 128, 128)], sem)

    def wait_gathers(w_v, s_v, sem):
        for j in range(KI):
            pltpu.make_async_copy(wp_hbm.at[ix2_v.at[pl.ds(j * 128, 128)]],
                                  w_v.at[pl.ds(j * 128, 128)], sem).wait()
            pltpu.make_async_copy(sp_hbm.at[ix16_v.at[pl.ds(j * 128, 128)]],
                                  s_v.at[pl.ds(j * 128, 128)], sem).wait()

    def out_slice(c):
        return out_hbm.at[pl.ds(pl.multiple_of((base + c * K) * D, 8), K * D)]

    def compute(c_half, w_v, s_v, o_v):
        def blk(t, _):
            iv = ixc_v[pl.ds(c_half * K + t * L, L)]
            p8 = (iv & 1) << 3
            mv = iv & 15
            for i in range(L):
                r = t * L + i
                wrow = w_v[r] ^ jnp.int32(_XOR8)
                sc = dgather(s_v[r], jnp.full((L,), mv[i], jnp.int32))
                wb = wsel + p8[i]
                for v in range(4):
                    q = lax.shift_right_arithmetic(
                        lax.shift_left(dgather(wrow, wb + 2 * v), shl), 28)
                    o_v[pl.ds(r * D + v * L, L)] = q.astype(jnp.float32) * sc
            return 0
        lax.fori_loop(0, K // L, blk, 0)

    stage_pair(0)
    fire_gathers(0, wA_v, sA_v, semA)
    fire_gathers(1, wB_v, sB_v, semB)
    pltpu.async_copy(oA_v, out_slice(0), semOA)
    pltpu.async_copy(oB_v, out_slice(1), semOB)

    def body(t, _):
        a = 2 * t
        wait_gathers(wA_v, sA_v, semA)
        wait_gathers(wB_v, sB_v, semB)

        def cpb(i, _):
            ixc_v[pl.ds(i * L, L)] = ixn_v[pl.ds(i * L, L)]
            return 0
        lax.fori_loop(0, 2 * K // L, cpb, 0)
        stage_pair(t + 1)

        pltpu.make_async_copy(oA_v, out_slice(a), semOA).wait()
        compute(0, wA_v, sA_v, oA_v)
        pltpu.async_copy(oA_v, out_slice(a), semOA)
        fire_gathers(0, wA_v, sA_v, semA)

        pltpu.make_async_copy(oB_v, out_slice(a), semOB).wait()
        compute(1, wB_v, sB_v, oB_v)
        pltpu.async_copy(oB_v, out_slice(a + 1), semOB)
        fire_gathers(1, wB_v, sB_v, semB)
        return 0

    lax.fori_loop(0, NP, body, 0)

    wait_gathers(wA_v, sA_v, semA)
    wait_gathers(wB_v, sB_v, semB)
    pltpu.make_async_copy(oA_v, out_slice(0), semOA).wait()
    pltpu.make_async_copy(oB_v, out_slice(0), semOB).wait()


@jax.jit
def _lookup(xpad, wp, sp):
    mesh = plsc.VectorSubcoreMesh(core_axis_name="c", subcore_axis_name="s")
    run = functools.partial(
        pl.kernel,
        mesh=mesh,
        compiler_params=pltpu.CompilerParams(use_tc_tiling_on_sc=False),
        out_type=jax.ShapeDtypeStruct((B * D,), jnp.float32),
        scratch_types=[
            pltpu.VMEM((2 * K,), jnp.int32),
            pltpu.VMEM((2 * K,), jnp.int32),
            pltpu.VMEM((2 * K,), jnp.int32),
            pltpu.VMEM((2 * K,), jnp.int32),
            pltpu.VMEM((K, L), jnp.int32),
            pltpu.VMEM((K, L), jnp.int32),
            pltpu.VMEM((K, L), jnp.float32),
            pltpu.VMEM((K, L), jnp.float32),
            pltpu.VMEM((K * D,), jnp.float32),
            pltpu.VMEM((K * D,), jnp.float32),
            pltpu.SemaphoreType.DMA,
            pltpu.SemaphoreType.DMA,
            pltpu.SemaphoreType.DMA,
            pltpu.SemaphoreType.DMA,
        ],
    )(_sc_body)
    return run(xpad, wp, sp)


def kernel(x, weight, scales):
    xf = x.reshape(B)
    xpad = jnp.concatenate([xf, xf[:2 * K]])
    wp = weight.reshape(V // 2, 2 * 8)
    sp = scales.astype(jnp.float32).reshape(V // L, L)
    out = _lookup(xpad, wp, sp)
    return out.reshape(4096, 200, 64)
